# baseline (speedup 1.0000x reference)
import functools
import math

import jax
import jax.numpy as jnp
from jax import lax
from jax.experimental import pallas as pl
from jax.experimental.pallas import tpu as pltpu
from jax.experimental.pallas import tpu_sc as plsc

NC = 2
NS = 16
NW = NC * NS
L = 16


def _emb_body(scale, n_chunks, chunk, d_model,
              t_hbm, c_hbm, tok_hbm, comb_hbm, out_hbm,
              tv, cv,
              a0, b0, o0, a1, b1, o1,
              gs0, gs1, os0, os1):
    per_w = n_chunks * chunk
    cid = lax.axis_index("c")
    sid = lax.axis_index("s")
    wid = sid * NC + cid
    base = wid * per_w

    sets = ((a0, b0, o0, gs0, os0), (a1, b1, o1, gs1, os1))

    pltpu.sync_copy(t_hbm.at[pl.ds(base, per_w)], tv)
    pltpu.sync_copy(c_hbm.at[pl.ds(base, per_w)], cv)

    def issue_gathers(g, bufs):
        a, b, _, gsem, _ = bufs
        off = g * chunk
        pltpu.async_copy(tok_hbm.at[tv.at[pl.ds(off, chunk)]], a, gsem)
        pltpu.async_copy(comb_hbm.at[cv.at[pl.ds(off, chunk)]], b, gsem)

    issue_gathers(0, sets[0])
    issue_gathers(1, sets[1])

    @pl.loop(0, n_chunks, step=2)
    def _pipeline(i):
        for k in range(2):
            a, b, o, gsem, osem = sets[k]
            g = i + k
            off = g * chunk
            pltpu.make_async_copy(
                tok_hbm.at[tv.at[pl.ds(off, chunk)]], a, gsem).wait()
            pltpu.make_async_copy(
                comb_hbm.at[cv.at[pl.ds(off, chunk)]], b, gsem).wait()

            @pl.when(g >= 2)
            def _():
                pltpu.make_async_copy(
                    o, out_hbm.at[pl.ds(base + off, chunk)], osem).wait()

            def combine_row(r, c2):
                for j in range(d_model // L):
                    sl = pl.ds(j * L, L)
                    o[r, sl] = (a[r, sl] + b[r, sl]) * scale
                return c2

            lax.fori_loop(0, chunk, combine_row, 0)

            pltpu.async_copy(o, out_hbm.at[pl.ds(base + off, chunk)], osem)

            @pl.when(g + 2 < n_chunks)
            def _():
                issue_gathers(g + 2, sets[k])

    for k in range(2):
        o, osem = sets[k][2], sets[k][4]
        pltpu.make_async_copy(o, out_hbm.at[pl.ds(base, chunk)], osem).wait()


@jax.jit
def kernel(t, p, s, tok_w, pos_w, seg_w):
    b, s_len = t.shape
    d_model = tok_w.shape[1]
    n_seg = seg_w.shape[0]
    total = b * s_len
    scale = math.sqrt(float(d_model))

    chunk = 16
    assert total % (NW * chunk) == 0
    n_chunks = total // (NW * chunk)
    assert n_chunks % 2 == 0 and n_chunks >= 4

    tf = t.reshape(total).astype(jnp.int32)
    comb = jnp.concatenate([pos_w + seg_w[k] for k in range(n_seg)],
                           axis=1).reshape(-1, d_model)
    cf = (p.reshape(total).astype(jnp.int32) * n_seg
          + s.reshape(total).astype(jnp.int32))

    mesh = plsc.VectorSubcoreMesh(core_axis_name="c", subcore_axis_name="s",
                                  num_cores=NC, num_subcores=NS)
    body = functools.partial(_emb_body, scale, n_chunks, chunk, d_model)
    per_w = n_chunks * chunk
    buf = pltpu.VMEM((chunk, d_model), jnp.float32)
    run = pl.kernel(
        body,
        out_type=jax.ShapeDtypeStruct((total, d_model), jnp.float32),
        mesh=mesh,
        scratch_types=[
            pltpu.VMEM((per_w,), jnp.int32),
            pltpu.VMEM((per_w,), jnp.int32),
            buf, buf, buf, buf, buf, buf,
            pltpu.SemaphoreType.DMA,
            pltpu.SemaphoreType.DMA,
            pltpu.SemaphoreType.DMA,
            pltpu.SemaphoreType.DMA,
        ],
    )
    out = run(tf, cf, tok_w, comb)
    return out.reshape(b, s_len, d_model)


# speedup vs baseline: 1.2058x; 1.2058x over previous
import functools
import math

import jax
import jax.numpy as jnp
from jax import lax
from jax.experimental import pallas as pl
from jax.experimental.pallas import tpu as pltpu
from jax.experimental.pallas import tpu_sc as plsc

NC = 2
NS = 16
NW = NC * NS
L = 16


def _emb_body(scale, n_chunks, chunk, d_model,
              t_hbm, c_hbm, tok_hbm, comb_hbm, out_hbm,
              tv, cv,
              a0, b0, o0, a1, b1, o1,
              gs0, gs1, os0, os1):
    per_w = n_chunks * chunk
    cid = lax.axis_index("c")
    sid = lax.axis_index("s")
    wid = sid * NC + cid
    base = wid * per_w

    sets = ((a0, b0, o0, gs0, os0), (a1, b1, o1, gs1, os1))

    pltpu.sync_copy(t_hbm.at[pl.ds(base, per_w)], tv)
    pltpu.sync_copy(c_hbm.at[pl.ds(base, per_w)], cv)

    def issue_gathers(g, bufs):
        a, b, _, gsem, _ = bufs
        off = g * chunk
        pltpu.async_copy(tok_hbm.at[tv.at[pl.ds(off, chunk)]], a, gsem)
        pltpu.async_copy(comb_hbm.at[cv.at[pl.ds(off, chunk)]], b, gsem)

    issue_gathers(0, sets[0])
    issue_gathers(1, sets[1])

    @pl.loop(0, n_chunks, step=2)
    def _pipeline(i):
        for k in range(2):
            a, b, o, gsem, osem = sets[k]
            g = i + k
            off = g * chunk
            pltpu.make_async_copy(
                tok_hbm.at[tv.at[pl.ds(off, chunk)]], a, gsem).wait()
            pltpu.make_async_copy(
                comb_hbm.at[cv.at[pl.ds(off, chunk)]], b, gsem).wait()

            @pl.when(g >= 2)
            def _():
                pltpu.make_async_copy(
                    o, out_hbm.at[pl.ds(base + off, chunk)], osem).wait()

            def combine_row(r, c2):
                for j in range(d_model // L):
                    sl = pl.ds(j * L, L)
                    o[r, sl] = (a[r, sl] + b[r, sl]) * scale
                return c2

            lax.fori_loop(0, chunk, combine_row, 0)

            pltpu.async_copy(o, out_hbm.at[pl.ds(base + off, chunk)], osem)

            @pl.when(g + 2 < n_chunks)
            def _():
                issue_gathers(g + 2, sets[k])

    for k in range(2):
        o, osem = sets[k][2], sets[k][4]
        pltpu.make_async_copy(o, out_hbm.at[pl.ds(base, chunk)], osem).wait()


@jax.jit
def kernel(t, p, s, tok_w, pos_w, seg_w):
    b, s_len = t.shape
    d_model = tok_w.shape[1]
    n_seg = seg_w.shape[0]
    total = b * s_len
    scale = math.sqrt(float(d_model))

    chunk = 16
    assert total % (NW * chunk) == 0
    n_chunks = total // (NW * chunk)
    assert n_chunks % 2 == 0 and n_chunks >= 4

    tf = t.reshape(total).astype(jnp.int32)
    n_pos = pos_w.shape[0]
    comb = jnp.concatenate([pos_w + seg_w[k] for k in range(n_seg)], axis=0)
    cf = (s.reshape(total).astype(jnp.int32) * n_pos
          + p.reshape(total).astype(jnp.int32))

    mesh = plsc.VectorSubcoreMesh(core_axis_name="c", subcore_axis_name="s",
                                  num_cores=NC, num_subcores=NS)
    body = functools.partial(_emb_body, scale, n_chunks, chunk, d_model)
    per_w = n_chunks * chunk
    buf = pltpu.VMEM((chunk, d_model), jnp.float32)
    run = pl.kernel(
        body,
        out_type=jax.ShapeDtypeStruct((total, d_model), jnp.float32),
        mesh=mesh,
        scratch_types=[
            pltpu.VMEM((per_w,), jnp.int32),
            pltpu.VMEM((per_w,), jnp.int32),
            buf, buf, buf, buf, buf, buf,
            pltpu.SemaphoreType.DMA,
            pltpu.SemaphoreType.DMA,
            pltpu.SemaphoreType.DMA,
            pltpu.SemaphoreType.DMA,
        ],
    )
    out = run(tf, cf, tok_w, comb)
    return out.reshape(b, s_len, d_model)


# speedup vs baseline: 1.3334x; 1.1058x over previous
import functools
import math

import jax
import jax.numpy as jnp
from jax import lax
from jax.experimental import pallas as pl
from jax.experimental.pallas import tpu as pltpu
from jax.experimental.pallas import tpu_sc as plsc

NC = 2
NS = 16
NW = NC * NS
L = 16


def _comb_body(pos_ref, seg_ref, out_ref):
    si = pl.program_id(0)
    out_ref[...] = pos_ref[...] + seg_ref[pl.ds(si, 1), :]


def _build_comb(pos_w, seg_w):
    n_pos, d_model = pos_w.shape
    n_seg = seg_w.shape[0]
    br = 256
    grid = (n_seg, n_pos // br)
    return pl.pallas_call(
        _comb_body,
        grid=grid,
        in_specs=[
            pl.BlockSpec((br, d_model), lambda si, ri: (ri, 0)),
            pl.BlockSpec((n_seg, d_model), lambda si, ri: (0, 0)),
        ],
        out_specs=pl.BlockSpec((br, d_model),
                               lambda si, ri, _n=(n_pos // br): (si * _n + ri, 0)),
        out_shape=jax.ShapeDtypeStruct((n_seg * n_pos, d_model), jnp.float32),
    )(pos_w, seg_w)


def _emb_body(scale, n_chunks, chunk, d_model,
              t_hbm, c_hbm, tok_hbm, comb_hbm, out_hbm,
              tv, cv,
              a0, b0, o0, a1, b1, o1,
              gs0, gs1, os0, os1):
    per_w = n_chunks * chunk
    cid = lax.axis_index("c")
    sid = lax.axis_index("s")
    wid = sid * NC + cid
    base = wid * per_w

    sets = ((a0, b0, o0, gs0, os0), (a1, b1, o1, gs1, os1))

    pltpu.sync_copy(t_hbm.at[pl.ds(base, per_w)], tv)
    pltpu.sync_copy(c_hbm.at[pl.ds(base, per_w)], cv)

    def issue_gathers(g, bufs):
        a, b, _, gsem, _ = bufs
        off = g * chunk
        pltpu.async_copy(tok_hbm.at[tv.at[pl.ds(off, chunk)]], a, gsem)
        pltpu.async_copy(comb_hbm.at[cv.at[pl.ds(off, chunk)]], b, gsem)

    issue_gathers(0, sets[0])
    issue_gathers(1, sets[1])

    @pl.loop(0, n_chunks, step=2)
    def _pipeline(i):
        for k in range(2):
            a, b, o, gsem, osem = sets[k]
            g = i + k
            off = g * chunk
            pltpu.make_async_copy(
                tok_hbm.at[tv.at[pl.ds(off, chunk)]], a, gsem).wait()
            pltpu.make_async_copy(
                comb_hbm.at[cv.at[pl.ds(off, chunk)]], b, gsem).wait()

            @pl.when(g >= 2)
            def _():
                pltpu.make_async_copy(
                    o, out_hbm.at[pl.ds(base + off, chunk)], osem).wait()

            def combine_row(r, c2):
                for j in range(d_model // L):
                    sl = pl.ds(j * L, L)
                    o[r, sl] = (a[r, sl] + b[r, sl]) * scale
                return c2

            lax.fori_loop(0, chunk, combine_row, 0)

            pltpu.async_copy(o, out_hbm.at[pl.ds(base + off, chunk)], osem)

            @pl.when(g + 2 < n_chunks)
            def _():
                issue_gathers(g + 2, sets[k])

    for k in range(2):
        o, osem = sets[k][2], sets[k][4]
        pltpu.make_async_copy(o, out_hbm.at[pl.ds(base, chunk)], osem).wait()


@jax.jit
def kernel(t, p, s, tok_w, pos_w, seg_w):
    b, s_len = t.shape
    d_model = tok_w.shape[1]
    n_seg = seg_w.shape[0]
    total = b * s_len
    scale = math.sqrt(float(d_model))

    chunk = 16
    assert total % (NW * chunk) == 0
    n_chunks = total // (NW * chunk)
    assert n_chunks % 2 == 0 and n_chunks >= 4

    tf = t.reshape(total).astype(jnp.int32)
    n_pos = pos_w.shape[0]
    comb = _build_comb(pos_w, seg_w)
    cf = (s.reshape(total).astype(jnp.int32) * n_pos
          + p.reshape(total).astype(jnp.int32))

    mesh = plsc.VectorSubcoreMesh(core_axis_name="c", subcore_axis_name="s",
                                  num_cores=NC, num_subcores=NS)
    body = functools.partial(_emb_body, scale, n_chunks, chunk, d_model)
    per_w = n_chunks * chunk
    buf = pltpu.VMEM((chunk, d_model), jnp.float32)
    run = pl.kernel(
        body,
        out_type=jax.ShapeDtypeStruct((total, d_model), jnp.float32),
        mesh=mesh,
        scratch_types=[
            pltpu.VMEM((per_w,), jnp.int32),
            pltpu.VMEM((per_w,), jnp.int32),
            buf, buf, buf, buf, buf, buf,
            pltpu.SemaphoreType.DMA,
            pltpu.SemaphoreType.DMA,
            pltpu.SemaphoreType.DMA,
            pltpu.SemaphoreType.DMA,
        ],
    )
    out = run(tf, cf, tok_w, comb)
    return out.reshape(b, s_len, d_model)


# speedup vs baseline: 1.4093x; 1.0569x over previous
import functools
import math

import jax
import jax.numpy as jnp
from jax import lax
from jax.experimental import pallas as pl
from jax.experimental.pallas import tpu as pltpu
from jax.experimental.pallas import tpu_sc as plsc

NC = 2
NS = 16
NW = NC * NS
L = 16


def _comb_body(pos_ref, seg_ref, out_ref):
    si = pl.program_id(0)
    out_ref[...] = pos_ref[...] + seg_ref[pl.ds(si, 1), :]


def _build_comb(pos_w, seg_w):
    n_pos, d_model = pos_w.shape
    n_seg = seg_w.shape[0]
    br = 512
    grid = (n_seg, n_pos // br)
    return pl.pallas_call(
        _comb_body,
        grid=grid,
        in_specs=[
            pl.BlockSpec((br, d_model), lambda si, ri: (ri, 0)),
            pl.BlockSpec((n_seg, d_model), lambda si, ri: (0, 0)),
        ],
        out_specs=pl.BlockSpec((br, d_model),
                               lambda si, ri, _n=(n_pos // br): (si * _n + ri, 0)),
        out_shape=jax.ShapeDtypeStruct((n_seg * n_pos, d_model), jnp.float32),
    )(pos_w, seg_w)


def _emb_body(scale, n_chunks, chunk, d_model,
              t_hbm, c_hbm, tok_hbm, comb_hbm, out_hbm,
              tv, cv,
              a0, b0, o0, a1, b1, o1,
              gs0, gs1, os0, os1):
    per_w = n_chunks * chunk
    cid = lax.axis_index("c")
    sid = lax.axis_index("s")
    wid = sid * NC + cid
    base = wid * per_w

    sets = ((a0, b0, o0, gs0, os0), (a1, b1, o1, gs1, os1))

    pltpu.sync_copy(t_hbm.at[pl.ds(base, per_w)], tv)
    pltpu.sync_copy(c_hbm.at[pl.ds(base, per_w)], cv)

    def issue_gathers(g, bufs):
        a, b, _, gsem, _ = bufs
        off = g * chunk
        pltpu.async_copy(tok_hbm.at[tv.at[pl.ds(off, chunk)]], a, gsem)
        pltpu.async_copy(comb_hbm.at[cv.at[pl.ds(off, chunk)]], b, gsem)

    issue_gathers(0, sets[0])
    issue_gathers(1, sets[1])

    @pl.loop(0, n_chunks, step=2)
    def _pipeline(i):
        for k in range(2):
            a, b, o, gsem, osem = sets[k]
            g = i + k
            off = g * chunk
            pltpu.make_async_copy(
                tok_hbm.at[tv.at[pl.ds(off, chunk)]], a, gsem).wait()
            pltpu.make_async_copy(
                comb_hbm.at[cv.at[pl.ds(off, chunk)]], b, gsem).wait()

            @pl.when(g >= 2)
            def _():
                pltpu.make_async_copy(
                    o, out_hbm.at[pl.ds(base + off, chunk)], osem).wait()

            def combine_row(r, c2):
                for j in range(d_model // L):
                    sl = pl.ds(j * L, L)
                    o[r, sl] = (a[r, sl] + b[r, sl]) * scale
                return c2

            lax.fori_loop(0, chunk, combine_row, 0)

            pltpu.async_copy(o, out_hbm.at[pl.ds(base + off, chunk)], osem)

            @pl.when(g + 2 < n_chunks)
            def _():
                issue_gathers(g + 2, sets[k])

    for k in range(2):
        o, osem = sets[k][2], sets[k][4]
        pltpu.make_async_copy(o, out_hbm.at[pl.ds(base, chunk)], osem).wait()


@jax.jit
def kernel(t, p, s, tok_w, pos_w, seg_w):
    b, s_len = t.shape
    d_model = tok_w.shape[1]
    n_seg = seg_w.shape[0]
    total = b * s_len
    scale = math.sqrt(float(d_model))

    chunk = 16
    assert total % (NW * chunk) == 0
    n_chunks = total // (NW * chunk)
    assert n_chunks % 2 == 0 and n_chunks >= 4

    tf = t.reshape(total).astype(jnp.int32)
    n_pos = pos_w.shape[0]
    comb = _build_comb(pos_w, seg_w)
    cf = (s.reshape(total).astype(jnp.int32) * n_pos
          + p.reshape(total).astype(jnp.int32))

    mesh = plsc.VectorSubcoreMesh(core_axis_name="c", subcore_axis_name="s",
                                  num_cores=NC, num_subcores=NS)
    body = functools.partial(_emb_body, scale, n_chunks, chunk, d_model)
    per_w = n_chunks * chunk
    buf = pltpu.VMEM((chunk, d_model), jnp.float32)
    run = pl.kernel(
        body,
        out_type=jax.ShapeDtypeStruct((total, d_model), jnp.float32),
        mesh=mesh,
        scratch_types=[
            pltpu.VMEM((per_w,), jnp.int32),
            pltpu.VMEM((per_w,), jnp.int32),
            buf, buf, buf, buf, buf, buf,
            pltpu.SemaphoreType.DMA,
            pltpu.SemaphoreType.DMA,
            pltpu.SemaphoreType.DMA,
            pltpu.SemaphoreType.DMA,
        ],
    )
    out = run(tf, cf, tok_w, comb)
    return out.reshape(b, s_len, d_model)


# speedup vs baseline: 1.4253x; 1.0114x over previous
import functools
import math

import jax
import jax.numpy as jnp
from jax import lax
from jax.experimental import pallas as pl
from jax.experimental.pallas import tpu as pltpu
from jax.experimental.pallas import tpu_sc as plsc

NC = 2
NS = 16
NW = NC * NS
L = 16


def _comb_body(pos_ref, seg_ref, out_ref):
    si = pl.program_id(0)
    out_ref[...] = pos_ref[...] + seg_ref[pl.ds(si, 1), :]


def _build_comb(pos_w, seg_w):
    n_pos, d_model = pos_w.shape
    n_seg = seg_w.shape[0]
    br = 1024
    grid = (n_seg, n_pos // br)
    return pl.pallas_call(
        _comb_body,
        grid=grid,
        in_specs=[
            pl.BlockSpec((br, d_model), lambda si, ri: (ri, 0)),
            pl.BlockSpec((n_seg, d_model), lambda si, ri: (0, 0)),
        ],
        out_specs=pl.BlockSpec((br, d_model),
                               lambda si, ri, _n=(n_pos // br): (si * _n + ri, 0)),
        out_shape=jax.ShapeDtypeStruct((n_seg * n_pos, d_model), jnp.float32),
    )(pos_w, seg_w)


def _emb_body(scale, n_chunks, chunk, d_model,
              t_hbm, c_hbm, tok_hbm, comb_hbm, out_hbm,
              tv, cv,
              a0, b0, o0, a1, b1, o1,
              gs0, gs1, os0, os1):
    per_w = n_chunks * chunk
    cid = lax.axis_index("c")
    sid = lax.axis_index("s")
    wid = sid * NC + cid
    base = wid * per_w

    sets = ((a0, b0, o0, gs0, os0), (a1, b1, o1, gs1, os1))

    pltpu.sync_copy(t_hbm.at[pl.ds(base, per_w)], tv)
    pltpu.sync_copy(c_hbm.at[pl.ds(base, per_w)], cv)

    def issue_gathers(g, bufs):
        a, b, _, gsem, _ = bufs
        off = g * chunk
        pltpu.async_copy(tok_hbm.at[tv.at[pl.ds(off, chunk)]], a, gsem)
        pltpu.async_copy(comb_hbm.at[cv.at[pl.ds(off, chunk)]], b, gsem)

    issue_gathers(0, sets[0])
    issue_gathers(1, sets[1])

    @pl.loop(0, n_chunks, step=2)
    def _pipeline(i):
        for k in range(2):
            a, b, o, gsem, osem = sets[k]
            g = i + k
            off = g * chunk
            pltpu.make_async_copy(
                tok_hbm.at[tv.at[pl.ds(off, chunk)]], a, gsem).wait()
            pltpu.make_async_copy(
                comb_hbm.at[cv.at[pl.ds(off, chunk)]], b, gsem).wait()

            @pl.when(g >= 2)
            def _():
                pltpu.make_async_copy(
                    o, out_hbm.at[pl.ds(base + off, chunk)], osem).wait()

            def combine_row(r, c2):
                for j in range(d_model // L):
                    sl = pl.ds(j * L, L)
                    o[r, sl] = (a[r, sl] + b[r, sl]) * scale
                return c2

            lax.fori_loop(0, chunk, combine_row, 0)

            pltpu.async_copy(o, out_hbm.at[pl.ds(base + off, chunk)], osem)

            @pl.when(g + 2 < n_chunks)
            def _():
                issue_gathers(g + 2, sets[k])

    for k in range(2):
        o, osem = sets[k][2], sets[k][4]
        pltpu.make_async_copy(o, out_hbm.at[pl.ds(base, chunk)], osem).wait()


@jax.jit
def kernel(t, p, s, tok_w, pos_w, seg_w):
    b, s_len = t.shape
    d_model = tok_w.shape[1]
    n_seg = seg_w.shape[0]
    total = b * s_len
    scale = math.sqrt(float(d_model))

    chunk = 16
    assert total % (NW * chunk) == 0
    n_chunks = total // (NW * chunk)
    assert n_chunks % 2 == 0 and n_chunks >= 4

    tf = t.reshape(total).astype(jnp.int32)
    n_pos = pos_w.shape[0]
    comb = _build_comb(pos_w, seg_w)
    cf = (s.reshape(total).astype(jnp.int32) * n_pos
          + p.reshape(total).astype(jnp.int32))

    mesh = plsc.VectorSubcoreMesh(core_axis_name="c", subcore_axis_name="s",
                                  num_cores=NC, num_subcores=NS)
    body = functools.partial(_emb_body, scale, n_chunks, chunk, d_model)
    per_w = n_chunks * chunk
    buf = pltpu.VMEM((chunk, d_model), jnp.float32)
    run = pl.kernel(
        body,
        out_type=jax.ShapeDtypeStruct((total, d_model), jnp.float32),
        mesh=mesh,
        scratch_types=[
            pltpu.VMEM((per_w,), jnp.int32),
            pltpu.VMEM((per_w,), jnp.int32),
            buf, buf, buf, buf, buf, buf,
            pltpu.SemaphoreType.DMA,
            pltpu.SemaphoreType.DMA,
            pltpu.SemaphoreType.DMA,
            pltpu.SemaphoreType.DMA,
        ],
    )
    out = run(tf, cf, tok_w, comb)
    return out.reshape(b, s_len, d_model)
